# baseline (speedup 1.0000x reference)
import functools

import jax
import jax.numpy as jnp
from jax import lax
from jax.experimental import pallas as pl
from jax.experimental.pallas import tpu as pltpu
from jax.experimental.pallas import tpu_sc as plsc

C = 16
H = 1024
W = 1024
N = 1048576
LANES = 16
NUM_CORES = 2
NUM_SUBCORES = 16
NW = NUM_CORES * NUM_SUBCORES
PPW = N // NW
CHUNK = 512
NG16 = CHUNK // LANES
GROWS = 128
NGD = CHUNK // GROWS
NCHUNKS = PPW // CHUNK
NPAIRS = NCHUNKS // 2
GPP = GROWS // LANES


NT = CHUNK // 128


def _sc_body(xs_hbm, ys_hbm, table_hbm, out_hbm,
             xv, yv,
             ia0, ia1, ia2, ia3, ib0, ib1, ib2, ib3,
             wa0, wa1, wa2, wa3, wb0, wb1, wb2, wb3,
             ra0, ra1, ra2, ra3, rb0, rb1, rb2, rb3,
             oa, ob, sem_a, sem_b, sem_o):
    cid = lax.axis_index("c")
    sid = lax.axis_index("s")
    wid = sid * NUM_CORES + cid
    lane = lax.iota(jnp.int32, LANES)

    bufs = {
        0: ((ia0, ia1, ia2, ia3), (wa0, wa1, wa2, wa3),
            (ra0, ra1, ra2, ra3), sem_a, oa),
        1: ((ib0, ib1, ib2, ib3), (wb0, wb1, wb2, wb3),
            (rb0, rb1, rb2, rb3), sem_b, ob),
    }

    def drain_out(b, prev_chunk):
        out_v = bufs[b][4]
        base = wid * PPW + prev_chunk * CHUNK
        for cb in range(2):
            pltpu.make_async_copy(
                out_v.at[cb],
                out_hbm.at[cb, pl.ds(base // 128, NT)],
                sem_o).wait()

    def load_xy(pj):
        base = wid * PPW + pj * (2 * CHUNK)
        pltpu.sync_copy(xs_hbm.at[pl.ds(base, 2 * CHUNK)], xv)
        pltpu.sync_copy(ys_hbm.at[pl.ds(base, 2 * CHUNK)], yv)

    def fire(half, b):
        idxs, ws, rows, sem, _ = bufs[b]

        @plsc.parallel_loop(0, NG16, unroll=4)
        def grp(g):
            s_x = pl.ds(half * CHUNK + g * LANES, LANES)
            gx = xv[s_x]
            gy = yv[s_x]
            fx = gx * (W - 1.0)
            fy = gy * (H - 1.0)
            fx = jnp.minimum(jnp.maximum(fx, 0.0), W - 1.0)
            fy = jnp.minimum(jnp.maximum(fy, 0.0), H - 1.0)
            x0 = jnp.minimum(fx.astype(jnp.int32), W - 2)
            y0 = jnp.minimum(fy.astype(jnp.int32), H - 2)
            wx1 = fx - x0.astype(jnp.float32)
            wy1 = fy - y0.astype(jnp.float32)
            wx0 = 1.0 - wx1
            wy0 = 1.0 - wy1
            lin = y0 * W + x0
            part = g // GPP
            off = (g % GPP) * LANES
            idxs[0][part, pl.ds(off, LANES)] = lin
            idxs[1][part, pl.ds(off, LANES)] = lin + 1
            idxs[2][part, pl.ds(off, LANES)] = lin + W
            idxs[3][part, pl.ds(off, LANES)] = lin + (W + 1)
            s = pl.ds(g * LANES, LANES)
            ws[0][s] = wy0 * wx0
            ws[1][s] = wy0 * wx1
            ws[2][s] = wy1 * wx0
            ws[3][s] = wy1 * wx1

        for cn in range(4):
            for part in range(NGD):
                pltpu.async_copy(
                    table_hbm.at[idxs[cn].at[part]],
                    rows[cn].at[pl.ds(part * GROWS, GROWS)],
                    sem)

    def combine(chunk, b, do_drain):
        idxs, ws, rows, sem, out_v = bufs[b]
        for cn in range(4):
            for part in range(NGD):
                pltpu.make_async_copy(
                    table_hbm.at[idxs[cn].at[part]],
                    rows[cn].at[pl.ds(part * GROWS, GROWS)],
                    sem).wait()

        @pl.when(do_drain)
        def _():
            drain_out(b, chunk - 2)

        r0, r1, r2, r3 = rows
        w0, w1, w2, w3 = ws

        @plsc.parallel_loop(0, NG16, unroll=4)
        def grp(g):
            r = g * LANES + lane
            s = pl.ds(g * LANES, LANES)
            a00 = w0[s]
            a01 = w1[s]
            a10 = w2[s]
            a11 = w3[s]
            t = g // 8
            col = pl.ds((g % 8) * LANES, LANES)
            for ch in range(C):
                cc = jnp.full((LANES,), ch, jnp.int32)
                v = (a00 * plsc.load_gather(r0, [r, cc])
                     + a01 * plsc.load_gather(r1, [r, cc])
                     + a10 * plsc.load_gather(r2, [r, cc])
                     + a11 * plsc.load_gather(r3, [r, cc]))
                out_v[ch // 8, t, ch % 8, col] = v

        base = wid * PPW + chunk * CHUNK
        for cb in range(2):
            pltpu.async_copy(
                out_v.at[cb],
                out_hbm.at[cb, pl.ds(base // 128, NT)],
                sem_o)

    load_xy(0)
    fire(0, 0)

    def body2(j, carry):
        fire(1, 1)
        combine(2 * j, 0, j >= 1)

        @pl.when(j < NPAIRS - 1)
        def _():
            load_xy(j + 1)
            fire(0, 0)

        combine(2 * j + 1, 1, j >= 1)
        return carry

    lax.fori_loop(0, NPAIRS, body2, 0)
    drain_out(0, NCHUNKS - 2)
    drain_out(1, NCHUNKS - 1)


@jax.jit
def _sc_sample(xs, ys, table):
    mesh = plsc.VectorSubcoreMesh(core_axis_name="c", subcore_axis_name="s")
    idx_t = pltpu.VMEM((NGD, GROWS), jnp.int32)
    w_t = pltpu.VMEM((CHUNK,), jnp.float32)
    row_t = pltpu.VMEM((CHUNK, C), jnp.float32)
    f = functools.partial(
        pl.kernel, mesh=mesh,
        out_type=jax.ShapeDtypeStruct((2, N // 128, 8, 128), jnp.float32),
        compiler_params=pltpu.CompilerParams(
            needs_layout_passes=False, use_tc_tiling_on_sc=False),
        scratch_types=(
            [pltpu.VMEM((2 * CHUNK,), jnp.float32)] * 2
            + [idx_t] * 8
            + [w_t] * 8
            + [row_t] * 8
            + [pltpu.VMEM((2, NT, 8, 128), jnp.float32)] * 2
            + [pltpu.SemaphoreType.DMA] * 3
        ),
    )(_sc_body)
    return f(xs, ys, table)


def kernel(ray, grid):
    assert ray.shape == (N, 2) and grid.shape == (1, C, H, W)
    xs = ray[:, 1]
    ys = ray[:, 0]
    table = grid[0].transpose(1, 2, 0).reshape(H * W, C)
    out_t = _sc_sample(xs, ys, table)
    return out_t.transpose(1, 3, 0, 2).reshape(N, C)


# speedup vs baseline: 1.1414x; 1.1414x over previous
import functools

import jax
import jax.numpy as jnp
from jax import lax
from jax.experimental import pallas as pl
from jax.experimental.pallas import tpu as pltpu
from jax.experimental.pallas import tpu_sc as plsc

C = 16
H = 1024
W = 1024
N = 1048576
LANES = 16
NUM_CORES = 2
NUM_SUBCORES = 16
NW = NUM_CORES * NUM_SUBCORES
PPW = N // NW
CHUNK = 512
NG16 = CHUNK // LANES
GROWS = 128
NGD = CHUNK // GROWS
NCHUNKS = PPW // CHUNK
NPAIRS = NCHUNKS // 2
GPP = GROWS // LANES


NT = CHUNK // 128


def _sc_body(xs_hbm, ys_hbm, table_hbm, out_hbm,
             xv, yv,
             ia0, ia1, ia2, ia3, ib0, ib1, ib2, ib3,
             wa0, wa1, wa2, wa3, wb0, wb1, wb2, wb3,
             ra0, ra1, ra2, ra3, rb0, rb1, rb2, rb3,
             oa, ob, sem_a, sem_b, sem_o):
    cid = lax.axis_index("c")
    sid = lax.axis_index("s")
    wid = sid * NUM_CORES + cid
    lane = lax.iota(jnp.int32, LANES)

    bufs = {
        0: ((ia0, ia1, ia2, ia3), (wa0, wa1, wa2, wa3),
            (ra0, ra1, ra2, ra3), sem_a, oa),
        1: ((ib0, ib1, ib2, ib3), (wb0, wb1, wb2, wb3),
            (rb0, rb1, rb2, rb3), sem_b, ob),
    }

    def drain_out(b, prev_chunk):
        out_v = bufs[b][4]
        base = wid * PPW + prev_chunk * CHUNK
        for cb in range(2):
            pltpu.make_async_copy(
                out_v.at[cb],
                out_hbm.at[cb, pl.ds(base // 128, NT)],
                sem_o).wait()

    def load_xy(pj):
        base = wid * PPW + pj * (2 * CHUNK)
        pltpu.sync_copy(xs_hbm.at[pl.ds(base, 2 * CHUNK)], xv)
        pltpu.sync_copy(ys_hbm.at[pl.ds(base, 2 * CHUNK)], yv)

    def fire(half, b):
        idxs, ws, rows, sem, _ = bufs[b]

        @plsc.parallel_loop(0, NG16, unroll=2)
        def grp(g):
            s_x = pl.ds(half * CHUNK + g * LANES, LANES)
            gx = xv[s_x]
            gy = yv[s_x]
            fx = gx * (W - 1.0)
            fy = gy * (H - 1.0)
            fx = jnp.minimum(jnp.maximum(fx, 0.0), W - 1.0)
            fy = jnp.minimum(jnp.maximum(fy, 0.0), H - 1.0)
            x0 = jnp.minimum(fx.astype(jnp.int32), W - 2)
            y0 = jnp.minimum(fy.astype(jnp.int32), H - 2)
            wx1 = fx - x0.astype(jnp.float32)
            wy1 = fy - y0.astype(jnp.float32)
            wx0 = 1.0 - wx1
            wy0 = 1.0 - wy1
            lin = y0 * W + x0
            part = g // GPP
            off = (g % GPP) * LANES
            idxs[0][part, pl.ds(off, LANES)] = lin
            idxs[1][part, pl.ds(off, LANES)] = lin + 1
            idxs[2][part, pl.ds(off, LANES)] = lin + W
            idxs[3][part, pl.ds(off, LANES)] = lin + (W + 1)
            s = pl.ds(g * LANES, LANES)
            ws[0][s] = wy0 * wx0
            ws[1][s] = wy0 * wx1
            ws[2][s] = wy1 * wx0
            ws[3][s] = wy1 * wx1

        for cn in range(4):
            for part in range(NGD):
                pltpu.async_copy(
                    table_hbm.at[idxs[cn].at[part]],
                    rows[cn].at[pl.ds(part * GROWS, GROWS)],
                    sem)

    def combine(chunk, b, do_drain):
        idxs, ws, rows, sem, out_v = bufs[b]
        for cn in range(4):
            for part in range(NGD):
                pltpu.make_async_copy(
                    table_hbm.at[idxs[cn].at[part]],
                    rows[cn].at[pl.ds(part * GROWS, GROWS)],
                    sem).wait()

        @pl.when(do_drain)
        def _():
            drain_out(b, chunk - 2)

        r0, r1, r2, r3 = rows
        w0, w1, w2, w3 = ws

        @plsc.parallel_loop(0, NG16, unroll=2)
        def grp(g):
            r = g * LANES + lane
            s = pl.ds(g * LANES, LANES)
            a00 = w0[s]
            a01 = w1[s]
            a10 = w2[s]
            a11 = w3[s]
            t = g // 8
            col = pl.ds((g % 8) * LANES, LANES)
            for ch in range(C):
                cc = jnp.full((LANES,), ch, jnp.int32)
                v = (a00 * plsc.load_gather(r0, [r, cc])
                     + a01 * plsc.load_gather(r1, [r, cc])
                     + a10 * plsc.load_gather(r2, [r, cc])
                     + a11 * plsc.load_gather(r3, [r, cc]))
                out_v[ch // 8, t, ch % 8, col] = v

        base = wid * PPW + chunk * CHUNK
        for cb in range(2):
            pltpu.async_copy(
                out_v.at[cb],
                out_hbm.at[cb, pl.ds(base // 128, NT)],
                sem_o)

    load_xy(0)
    fire(0, 0)

    def body2(j, carry):
        fire(1, 1)
        combine(2 * j, 0, j >= 1)

        @pl.when(j < NPAIRS - 1)
        def _():
            load_xy(j + 1)
            fire(0, 0)

        combine(2 * j + 1, 1, j >= 1)
        return carry

    lax.fori_loop(0, NPAIRS, body2, 0)
    drain_out(0, NCHUNKS - 2)
    drain_out(1, NCHUNKS - 1)


@jax.jit
def _sc_sample(xs, ys, table):
    mesh = plsc.VectorSubcoreMesh(core_axis_name="c", subcore_axis_name="s")
    idx_t = pltpu.VMEM((NGD, GROWS), jnp.int32)
    w_t = pltpu.VMEM((CHUNK,), jnp.float32)
    row_t = pltpu.VMEM((CHUNK, C), jnp.float32)
    f = functools.partial(
        pl.kernel, mesh=mesh,
        out_type=jax.ShapeDtypeStruct((2, N // 128, 8, 128), jnp.float32),
        compiler_params=pltpu.CompilerParams(
            needs_layout_passes=False, use_tc_tiling_on_sc=False),
        scratch_types=(
            [pltpu.VMEM((2 * CHUNK,), jnp.float32)] * 2
            + [idx_t] * 8
            + [w_t] * 8
            + [row_t] * 8
            + [pltpu.VMEM((2, NT, 8, 128), jnp.float32)] * 2
            + [pltpu.SemaphoreType.DMA] * 3
        ),
    )(_sc_body)
    return f(xs, ys, table)


def kernel(ray, grid):
    assert ray.shape == (N, 2) and grid.shape == (1, C, H, W)
    xs = ray[:, 1]
    ys = ray[:, 0]
    table = grid[0].transpose(1, 2, 0).reshape(H * W, C)
    out_t = _sc_sample(xs, ys, table)
    return out_t.transpose(1, 3, 0, 2).reshape(N, C)


# speedup vs baseline: 1.1459x; 1.0040x over previous
import functools

import jax
import jax.numpy as jnp
from jax import lax
from jax.experimental import pallas as pl
from jax.experimental.pallas import tpu as pltpu
from jax.experimental.pallas import tpu_sc as plsc

C = 16
H = 1024
W = 1024
N = 1048576
LANES = 16
NUM_CORES = 2
NUM_SUBCORES = 16
NW = NUM_CORES * NUM_SUBCORES
PPW = N // NW
CHUNK = 512
NG16 = CHUNK // LANES
GROWS = 512
NGD = CHUNK // GROWS
NCHUNKS = PPW // CHUNK
NPAIRS = NCHUNKS // 2
GPP = GROWS // LANES


NT = CHUNK // 128


def _sc_body(xs_hbm, ys_hbm, table_hbm, out_hbm,
             xv, yv,
             ia0, ia1, ia2, ia3, ib0, ib1, ib2, ib3,
             wa0, wa1, wa2, wa3, wb0, wb1, wb2, wb3,
             ra0, ra1, ra2, ra3, rb0, rb1, rb2, rb3,
             oa, ob, sem_a, sem_b, sem_o):
    cid = lax.axis_index("c")
    sid = lax.axis_index("s")
    wid = sid * NUM_CORES + cid
    lane = lax.iota(jnp.int32, LANES)

    bufs = {
        0: ((ia0, ia1, ia2, ia3), (wa0, wa1, wa2, wa3),
            (ra0, ra1, ra2, ra3), sem_a, oa),
        1: ((ib0, ib1, ib2, ib3), (wb0, wb1, wb2, wb3),
            (rb0, rb1, rb2, rb3), sem_b, ob),
    }

    def drain_out(b, prev_chunk):
        out_v = bufs[b][4]
        base = wid * PPW + prev_chunk * CHUNK
        for cb in range(2):
            pltpu.make_async_copy(
                out_v.at[cb],
                out_hbm.at[cb, pl.ds(base // 128, NT)],
                sem_o).wait()

    def load_xy(pj):
        base = wid * PPW + pj * (2 * CHUNK)
        pltpu.sync_copy(xs_hbm.at[pl.ds(base, 2 * CHUNK)], xv)
        pltpu.sync_copy(ys_hbm.at[pl.ds(base, 2 * CHUNK)], yv)

    def fire(half, b):
        idxs, ws, rows, sem, _ = bufs[b]

        @plsc.parallel_loop(0, NG16, unroll=2)
        def grp(g):
            s_x = pl.ds(half * CHUNK + g * LANES, LANES)
            gx = xv[s_x]
            gy = yv[s_x]
            fx = gx * (W - 1.0)
            fy = gy * (H - 1.0)
            fx = jnp.minimum(jnp.maximum(fx, 0.0), W - 1.0)
            fy = jnp.minimum(jnp.maximum(fy, 0.0), H - 1.0)
            x0 = jnp.minimum(fx.astype(jnp.int32), W - 2)
            y0 = jnp.minimum(fy.astype(jnp.int32), H - 2)
            wx1 = fx - x0.astype(jnp.float32)
            wy1 = fy - y0.astype(jnp.float32)
            wx0 = 1.0 - wx1
            wy0 = 1.0 - wy1
            lin = y0 * W + x0
            part = g // GPP
            off = (g % GPP) * LANES
            idxs[0][part, pl.ds(off, LANES)] = lin
            idxs[1][part, pl.ds(off, LANES)] = lin + 1
            idxs[2][part, pl.ds(off, LANES)] = lin + W
            idxs[3][part, pl.ds(off, LANES)] = lin + (W + 1)
            s = pl.ds(g * LANES, LANES)
            ws[0][s] = wy0 * wx0
            ws[1][s] = wy0 * wx1
            ws[2][s] = wy1 * wx0
            ws[3][s] = wy1 * wx1

        for cn in range(4):
            for part in range(NGD):
                pltpu.async_copy(
                    table_hbm.at[idxs[cn].at[part]],
                    rows[cn].at[pl.ds(part * GROWS, GROWS)],
                    sem)

    def combine(chunk, b, do_drain):
        idxs, ws, rows, sem, out_v = bufs[b]
        for cn in range(4):
            for part in range(NGD):
                pltpu.make_async_copy(
                    table_hbm.at[idxs[cn].at[part]],
                    rows[cn].at[pl.ds(part * GROWS, GROWS)],
                    sem).wait()

        @pl.when(do_drain)
        def _():
            drain_out(b, chunk - 2)

        r0, r1, r2, r3 = rows
        w0, w1, w2, w3 = ws

        @plsc.parallel_loop(0, NG16, unroll=2)
        def grp(g):
            r = g * LANES + lane
            s = pl.ds(g * LANES, LANES)
            a00 = w0[s]
            a01 = w1[s]
            a10 = w2[s]
            a11 = w3[s]
            t = g // 8
            col = pl.ds((g % 8) * LANES, LANES)
            for ch in range(C):
                cc = jnp.full((LANES,), ch, jnp.int32)
                v = (a00 * plsc.load_gather(r0, [r, cc])
                     + a01 * plsc.load_gather(r1, [r, cc])
                     + a10 * plsc.load_gather(r2, [r, cc])
                     + a11 * plsc.load_gather(r3, [r, cc]))
                out_v[ch // 8, t, ch % 8, col] = v

        base = wid * PPW + chunk * CHUNK
        for cb in range(2):
            pltpu.async_copy(
                out_v.at[cb],
                out_hbm.at[cb, pl.ds(base // 128, NT)],
                sem_o)

    load_xy(0)
    fire(0, 0)

    def body2(j, carry):
        fire(1, 1)
        combine(2 * j, 0, j >= 1)

        @pl.when(j < NPAIRS - 1)
        def _():
            load_xy(j + 1)
            fire(0, 0)

        combine(2 * j + 1, 1, j >= 1)
        return carry

    lax.fori_loop(0, NPAIRS, body2, 0)
    drain_out(0, NCHUNKS - 2)
    drain_out(1, NCHUNKS - 1)


@jax.jit
def _sc_sample(xs, ys, table):
    mesh = plsc.VectorSubcoreMesh(core_axis_name="c", subcore_axis_name="s")
    idx_t = pltpu.VMEM((NGD, GROWS), jnp.int32)
    w_t = pltpu.VMEM((CHUNK,), jnp.float32)
    row_t = pltpu.VMEM((CHUNK, C), jnp.float32)
    f = functools.partial(
        pl.kernel, mesh=mesh,
        out_type=jax.ShapeDtypeStruct((2, N // 128, 8, 128), jnp.float32),
        compiler_params=pltpu.CompilerParams(
            needs_layout_passes=False, use_tc_tiling_on_sc=False),
        scratch_types=(
            [pltpu.VMEM((2 * CHUNK,), jnp.float32)] * 2
            + [idx_t] * 8
            + [w_t] * 8
            + [row_t] * 8
            + [pltpu.VMEM((2, NT, 8, 128), jnp.float32)] * 2
            + [pltpu.SemaphoreType.DMA] * 3
        ),
    )(_sc_body)
    return f(xs, ys, table)


def kernel(ray, grid):
    assert ray.shape == (N, 2) and grid.shape == (1, C, H, W)
    xs = ray[:, 1]
    ys = ray[:, 0]
    table = grid[0].transpose(1, 2, 0).reshape(H * W, C)
    out_t = _sc_sample(xs, ys, table)
    return out_t.transpose(1, 3, 0, 2).reshape(N, C)


# speedup vs baseline: 1.1863x; 1.0352x over previous
import functools

import jax
import jax.numpy as jnp
from jax import lax
from jax.experimental import pallas as pl
from jax.experimental.pallas import tpu as pltpu
from jax.experimental.pallas import tpu_sc as plsc

C = 16
H = 1024
W = 1024
N = 1048576
LANES = 16
NUM_CORES = 2
NUM_SUBCORES = 16
NW = NUM_CORES * NUM_SUBCORES
PPW = N // NW
CHUNK = 512
NG16 = CHUNK // LANES
GROWS = 512
NGD = CHUNK // GROWS
NCHUNKS = PPW // CHUNK
NPAIRS = NCHUNKS // 2
GPP = GROWS // LANES


NT = CHUNK // 128


def _sc_body(xs_hbm, ys_hbm, table_hbm, out_hbm,
             xv, yv,
             ia0, ia1, ia2, ia3, ib0, ib1, ib2, ib3,
             wa0, wa1, wa2, wa3, wb0, wb1, wb2, wb3,
             ra0, ra1, ra2, ra3, rb0, rb1, rb2, rb3,
             oa, ob, sem_a, sem_b, sem_o):
    cid = lax.axis_index("c")
    sid = lax.axis_index("s")
    wid = sid * NUM_CORES + cid
    lane = lax.iota(jnp.int32, LANES)

    bufs = {
        0: ((ia0, ia1, ia2, ia3), (wa0, wa1, wa2, wa3),
            (ra0, ra1, ra2, ra3), sem_a, oa),
        1: ((ib0, ib1, ib2, ib3), (wb0, wb1, wb2, wb3),
            (rb0, rb1, rb2, rb3), sem_b, ob),
    }

    def drain_out(b, prev_chunk):
        out_v = bufs[b][4]
        base = wid * PPW + prev_chunk * CHUNK
        for cb in range(2):
            pltpu.make_async_copy(
                out_v.at[cb],
                out_hbm.at[cb, pl.ds(base // 128, NT)],
                sem_o).wait()

    def load_xy(q):
        base = wid * PPW + q * (8 * CHUNK)
        pltpu.sync_copy(xs_hbm.at[pl.ds(base, 8 * CHUNK)], xv)
        pltpu.sync_copy(ys_hbm.at[pl.ds(base, 8 * CHUNK)], yv)

    def fire(xoff, b):
        idxs, ws, rows, sem, _ = bufs[b]

        @plsc.parallel_loop(0, NG16, unroll=2)
        def grp(g):
            s_x = pl.ds(xoff + g * LANES, LANES)
            gx = xv[s_x]
            gy = yv[s_x]
            fx = gx * (W - 1.0)
            fy = gy * (H - 1.0)
            fx = jnp.minimum(jnp.maximum(fx, 0.0), W - 1.0)
            fy = jnp.minimum(jnp.maximum(fy, 0.0), H - 1.0)
            x0 = jnp.minimum(fx.astype(jnp.int32), W - 2)
            y0 = jnp.minimum(fy.astype(jnp.int32), H - 2)
            wx1 = fx - x0.astype(jnp.float32)
            wy1 = fy - y0.astype(jnp.float32)
            wx0 = 1.0 - wx1
            wy0 = 1.0 - wy1
            lin = y0 * W + x0
            part = g // GPP
            off = (g % GPP) * LANES
            idxs[0][part, pl.ds(off, LANES)] = lin
            idxs[1][part, pl.ds(off, LANES)] = lin + 1
            idxs[2][part, pl.ds(off, LANES)] = lin + W
            idxs[3][part, pl.ds(off, LANES)] = lin + (W + 1)
            s = pl.ds(g * LANES, LANES)
            ws[0][s] = wy0 * wx0
            ws[1][s] = wy0 * wx1
            ws[2][s] = wy1 * wx0
            ws[3][s] = wy1 * wx1

        for cn in range(4):
            for part in range(NGD):
                pltpu.async_copy(
                    table_hbm.at[idxs[cn].at[part]],
                    rows[cn].at[pl.ds(part * GROWS, GROWS)],
                    sem)

    def combine(chunk, b, do_drain):
        idxs, ws, rows, sem, out_v = bufs[b]
        for cn in range(4):
            for part in range(NGD):
                pltpu.make_async_copy(
                    table_hbm.at[idxs[cn].at[part]],
                    rows[cn].at[pl.ds(part * GROWS, GROWS)],
                    sem).wait()

        @pl.when(do_drain)
        def _():
            drain_out(b, chunk - 2)

        r0, r1, r2, r3 = rows
        w0, w1, w2, w3 = ws

        @plsc.parallel_loop(0, NG16, unroll=2)
        def grp(g):
            r = g * LANES + lane
            s = pl.ds(g * LANES, LANES)
            a00 = w0[s]
            a01 = w1[s]
            a10 = w2[s]
            a11 = w3[s]
            t = g // 8
            col = pl.ds((g % 8) * LANES, LANES)
            for ch in range(C):
                cc = jnp.full((LANES,), ch, jnp.int32)
                v = (a00 * plsc.load_gather(r0, [r, cc])
                     + a01 * plsc.load_gather(r1, [r, cc])
                     + a10 * plsc.load_gather(r2, [r, cc])
                     + a11 * plsc.load_gather(r3, [r, cc]))
                out_v[ch // 8, t, ch % 8, col] = v

        base = wid * PPW + chunk * CHUNK
        for cb in range(2):
            pltpu.async_copy(
                out_v.at[cb],
                out_hbm.at[cb, pl.ds(base // 128, NT)],
                sem_o)

    load_xy(0)
    fire(0, 0)

    def body2(j, carry):
        fire(((2 * j + 1) % 8) * CHUNK, 1)
        combine(2 * j, 0, j >= 1)

        @pl.when(j < NPAIRS - 1)
        def _():
            @pl.when((j + 1) % 4 == 0)
            def _():
                load_xy((j + 1) // 4)

            fire(((2 * j + 2) % 8) * CHUNK, 0)

        combine(2 * j + 1, 1, j >= 1)
        return carry

    lax.fori_loop(0, NPAIRS, body2, 0)
    drain_out(0, NCHUNKS - 2)
    drain_out(1, NCHUNKS - 1)


@jax.jit
def _sc_sample(xs, ys, table):
    mesh = plsc.VectorSubcoreMesh(core_axis_name="c", subcore_axis_name="s")
    idx_t = pltpu.VMEM((NGD, GROWS), jnp.int32)
    w_t = pltpu.VMEM((CHUNK,), jnp.float32)
    row_t = pltpu.VMEM((CHUNK, C), jnp.float32)
    f = functools.partial(
        pl.kernel, mesh=mesh,
        out_type=jax.ShapeDtypeStruct((2, N // 128, 8, 128), jnp.float32),
        compiler_params=pltpu.CompilerParams(
            needs_layout_passes=False, use_tc_tiling_on_sc=False),
        scratch_types=(
            [pltpu.VMEM((8 * CHUNK,), jnp.float32)] * 2
            + [idx_t] * 8
            + [w_t] * 8
            + [row_t] * 8
            + [pltpu.VMEM((2, NT, 8, 128), jnp.float32)] * 2
            + [pltpu.SemaphoreType.DMA] * 3
        ),
    )(_sc_body)
    return f(xs, ys, table)


def kernel(ray, grid):
    assert ray.shape == (N, 2) and grid.shape == (1, C, H, W)
    xs = ray[:, 1]
    ys = ray[:, 0]
    table = grid[0].transpose(1, 2, 0).reshape(H * W, C)
    out_t = _sc_sample(xs, ys, table)
    return out_t.transpose(1, 3, 0, 2).reshape(N, C)
